# baseline (speedup 1.0000x reference)
import functools

import jax
import jax.numpy as jnp
from jax import lax
from jax.experimental import pallas as pl
from jax.experimental.pallas import tpu as pltpu
from jax.experimental.pallas import tpu_sc as plsc

BATCH = 16384
EMBED_DIM = 32
NUM_CORES = 2
NUM_SUBCORES = 16
NUM_WORKERS = NUM_CORES * NUM_SUBCORES
BPW = BATCH // NUM_WORKERS


def _embed_body(g_hbm, a_hbm, o_hbm, z_hbm, wg_hbm, wa_hbm, wo_hbm, wz_hbm,
                out_hbm, gi, ai, oi, zi, gr, ar, cr, zr, sem):
    wid = lax.axis_index("s") * NUM_CORES + lax.axis_index("c")
    base = wid * BPW
    pltpu.sync_copy(g_hbm.at[pl.ds(base, BPW)], gi)
    pltpu.sync_copy(a_hbm.at[pl.ds(base, BPW)], ai)
    pltpu.sync_copy(o_hbm.at[pl.ds(base, BPW)], oi)
    pltpu.sync_copy(z_hbm.at[pl.ds(base, BPW)], zi)
    c0 = pltpu.async_copy(wg_hbm.at[gi], gr, sem)
    c1 = pltpu.async_copy(wa_hbm.at[ai], ar, sem)
    c2 = pltpu.async_copy(wo_hbm.at[oi], cr, sem)
    c3 = pltpu.async_copy(wz_hbm.at[zi], zr, sem)
    c0.wait()
    c1.wait()
    c2.wait()
    c3.wait()
    pltpu.sync_copy(gr, out_hbm.at[pl.ds(base, BPW), pl.ds(0, EMBED_DIM)])
    pltpu.sync_copy(ar, out_hbm.at[pl.ds(base, BPW), pl.ds(EMBED_DIM, EMBED_DIM)])
    pltpu.sync_copy(cr, out_hbm.at[pl.ds(base, BPW), pl.ds(2 * EMBED_DIM, EMBED_DIM)])
    pltpu.sync_copy(zr, out_hbm.at[pl.ds(base, BPW), pl.ds(3 * EMBED_DIM, EMBED_DIM)])


@jax.jit
def _embed(gender_idx, age_idx, occupation_idx, area_idx,
           W_gender, W_age, W_occupation, W_area):
    mesh = plsc.VectorSubcoreMesh(core_axis_name="c", subcore_axis_name="s")
    k = functools.partial(
        pl.kernel,
        mesh=mesh,
        out_type=jax.ShapeDtypeStruct((BATCH, 4 * EMBED_DIM), jnp.float32),
        scratch_types=[
            pltpu.VMEM((BPW,), jnp.int32),
            pltpu.VMEM((BPW,), jnp.int32),
            pltpu.VMEM((BPW,), jnp.int32),
            pltpu.VMEM((BPW,), jnp.int32),
            pltpu.VMEM((BPW, EMBED_DIM), jnp.float32),
            pltpu.VMEM((BPW, EMBED_DIM), jnp.float32),
            pltpu.VMEM((BPW, EMBED_DIM), jnp.float32),
            pltpu.VMEM((BPW, EMBED_DIM), jnp.float32),
            pltpu.SemaphoreType.DMA,
        ],
        compiler_params=pltpu.CompilerParams(use_tc_tiling_on_sc=False),
    )(_embed_body)
    return k(gender_idx, age_idx, occupation_idx, area_idx,
             W_gender, W_age, W_occupation, W_area)


def kernel(gender_idx, age_idx, occupation_idx, area_idx,
           W_gender, W_age, W_occupation, W_area):
    return _embed(gender_idx.astype(jnp.int32), age_idx.astype(jnp.int32),
                  occupation_idx.astype(jnp.int32), area_idx.astype(jnp.int32),
                  W_gender, W_age, W_occupation, W_area)


# speedup vs baseline: 1.2349x; 1.2349x over previous
import functools

import jax
import jax.numpy as jnp
import numpy as np
from jax import lax
from jax.experimental import pallas as pl
from jax.experimental.pallas import tpu as pltpu
from jax.experimental.pallas import tpu_sc as plsc

BATCH = 16384
D = 32
OUT_D = 4 * D
NUM_CORES = 2
NUM_SUBCORES = 16
NUM_WORKERS = NUM_CORES * NUM_SUBCORES
BPW = BATCH // NUM_WORKERS
L = 16
GROUPS = BPW // L
OFF_GENDER = 0
OFF_AGE = 2
OFF_OCC = 9
SMALL_ROWS = 30


def _embed_body(g_hbm, a_hbm, o_hbm, z_hbm, small_hbm, wz_hbm, out_hbm,
                gi, ai, oi, zi, small_v, zbuf, big, sem):
    wid = lax.axis_index("s") * NUM_CORES + lax.axis_index("c")
    base = wid * BPW
    pltpu.sync_copy(g_hbm.at[pl.ds(base, BPW)], gi)
    pltpu.sync_copy(a_hbm.at[pl.ds(base, BPW)], ai)
    pltpu.sync_copy(o_hbm.at[pl.ds(base, BPW)], oi)
    pltpu.sync_copy(z_hbm.at[pl.ds(base, BPW)], zi)
    pltpu.sync_copy(small_hbm, small_v)
    cz = pltpu.async_copy(wz_hbm.at[zi], zbuf, sem)

    iota = lax.iota(jnp.int32, L)

    def small_group(g, carry):
        row_off = (g * L + iota) * OUT_D
        for tbl, (buf, off) in enumerate(((gi, OFF_GENDER), (ai, OFF_AGE),
                                          (oi, OFF_OCC))):
            idxv = buf[pl.ds(g * L, L)]
            fb = (idxv + off) * D
            col0 = tbl * D
            for c in range(D):
                v = plsc.load_gather(small_v, [fb + c])
                plsc.store_scatter(big, [row_off + (col0 + c)], v)
        return carry

    lax.fori_loop(0, GROUPS, small_group, 0)

    cz.wait()

    def compact_row(r, carry):
        dst = r * OUT_D + 3 * D
        big[pl.ds(dst, L)] = zbuf[r, pl.ds(0, L)]
        big[pl.ds(dst + L, L)] = zbuf[r, pl.ds(L, L)]
        return carry

    lax.fori_loop(0, BPW, compact_row, 0)

    pltpu.sync_copy(big, out_hbm.at[pl.ds(base * OUT_D, BPW * OUT_D)])


@jax.jit
def _embed(gender_idx, age_idx, occupation_idx, area_idx, small_flat, W_area):
    mesh = plsc.VectorSubcoreMesh(core_axis_name="c", subcore_axis_name="s")
    k = functools.partial(
        pl.kernel,
        mesh=mesh,
        out_type=jax.ShapeDtypeStruct((BATCH * OUT_D,), jnp.float32),
        scratch_types=[
            pltpu.VMEM((BPW,), jnp.int32),
            pltpu.VMEM((BPW,), jnp.int32),
            pltpu.VMEM((BPW,), jnp.int32),
            pltpu.VMEM((BPW,), jnp.int32),
            pltpu.VMEM((SMALL_ROWS * D,), jnp.float32),
            pltpu.VMEM((BPW, D), jnp.float32),
            pltpu.VMEM((BPW * OUT_D,), jnp.float32),
            pltpu.SemaphoreType.DMA,
        ],
        compiler_params=pltpu.CompilerParams(use_tc_tiling_on_sc=False,
                                             needs_layout_passes=False),
    )(_embed_body)
    flat = k(gender_idx, age_idx, occupation_idx, area_idx, small_flat, W_area)
    return flat.reshape(BATCH, OUT_D)


def kernel(gender_idx, age_idx, occupation_idx, area_idx,
           W_gender, W_age, W_occupation, W_area):
    small_flat = jnp.concatenate(
        (W_gender, W_age, W_occupation), axis=0).reshape(-1)
    return _embed(gender_idx.astype(jnp.int32), age_idx.astype(jnp.int32),
                  occupation_idx.astype(jnp.int32), area_idx.astype(jnp.int32),
                  small_flat, W_area)
